# baseline (speedup 1.0000x reference)
import jax
import jax.numpy as jnp
from jax import lax
from jax.experimental import pallas as pl
from jax.experimental.pallas import tpu as pltpu
from jax.experimental.pallas import tpu_sc as plsc

_MAX_DEG = 10
_SC_BUCKETS = (1, 2, 3, 4, 5, 6, 7, 8)
_TC_BUCKETS = (9, 10)
_SC_R8 = 6400
_N_ATOMS = 100000
_N_FEAT = 128
_PER_DEG = 10000
_LANES = 16

_B = 8
_SG = 2
_GB = _B * _SG
_NW = 32
_GPW = 20
_SPAN = _GPW * _GB
_PAD_ROWS = _SPAN * _NW
_TAIL = _PER_DEG - (_NW - 1) * _SPAN


def _pool_body(atoms_hbm, *refs):
    nsc = len(_SC_BUCKETS)
    idx_hbms = refs[:nsc]
    out_hbm = refs[nsc]
    (idx0, idx1, gbuf0, gbuf1, osA, osB,
     sg0, sg1, sidx, sself, so0, so1) = refs[nsc + 1:]
    w = lax.axis_index("s") * 2 + lax.axis_index("c")

    idx_bufs = (idx0, idx1)
    gbufs = (gbuf0, gbuf1)
    os_bufs = (osA, osB)
    sgs = (sg0, sg1)
    so = (so0, so1)

    def idx_copy(pos, dst, sem):
        span_idx = _SPAN * _SC_BUCKETS[pos]
        pltpu.async_copy(
            idx_hbms[pos].at[pl.ds(w * span_idx, span_idx)],
            dst.at[pl.ds(0, span_idx)], sem)

    def idx_wait(pos, dst, sem):
        span_idx = _SPAN * _SC_BUCKETS[pos]
        pltpu.make_async_copy(
            idx_hbms[pos].at[pl.ds(0, span_idx)],
            dst.at[pl.ds(0, span_idx)], sem).wait()

    def self_copy(d, dst, sem):
        base_out = (d - 1) * _PER_DEG
        if d == 8:
            @pl.when(w < _SC_R8 // _SPAN)
            def _():
                o0 = pl.multiple_of(base_out + w * _SPAN, _B)
                pltpu.async_copy(atoms_hbm.at[pl.ds(o0, _SPAN), :], dst, sem)
            return

        @pl.when(w < _NW - 1)
        def _():
            o0 = pl.multiple_of(base_out + w * _SPAN, _B)
            pltpu.async_copy(atoms_hbm.at[pl.ds(o0, _SPAN), :], dst, sem)

        @pl.when(w == _NW - 1)
        def _():
            pltpu.async_copy(
                atoms_hbm.at[pl.ds(base_out + (_NW - 1) * _SPAN, _TAIL), :],
                dst.at[pl.ds(0, _TAIL)], sem)

    def self_wait(d, dst, sem):
        if d == 8:
            @pl.when(w < _SC_R8 // _SPAN)
            def _():
                pltpu.make_async_copy(atoms_hbm.at[pl.ds(0, _SPAN), :],
                                      dst, sem).wait()
            return

        @pl.when(w < _NW - 1)
        def _():
            pltpu.make_async_copy(atoms_hbm.at[pl.ds(0, _SPAN), :],
                                  dst, sem).wait()

        @pl.when(w == _NW - 1)
        def _():
            pltpu.make_async_copy(atoms_hbm.at[pl.ds(0, _TAIL), :],
                                  dst.at[pl.ds(0, _TAIL)], sem).wait()

    def out_write(pos, d, src, sem):
        base_out = pos * _PER_DEG
        if d == 8:
            @pl.when(w < _SC_R8 // _SPAN)
            def _():
                o0 = pl.multiple_of(base_out + w * _SPAN, _B)
                pltpu.async_copy(src, out_hbm.at[pl.ds(o0, _SPAN), :], sem)
            return

        @pl.when(w < _NW - 1)
        def _():
            o0 = pl.multiple_of(base_out + w * _SPAN, _B)
            pltpu.async_copy(src, out_hbm.at[pl.ds(o0, _SPAN), :], sem)

        @pl.when(w == _NW - 1)
        def _():
            pltpu.async_copy(
                src.at[pl.ds(0, _TAIL)],
                out_hbm.at[pl.ds(base_out + (_NW - 1) * _SPAN, _TAIL), :],
                sem)

    def out_wait(d, src, sem):
        if d == 8:
            @pl.when(w < _SC_R8 // _SPAN)
            def _():
                pltpu.make_async_copy(src, out_hbm.at[pl.ds(0, _SPAN), :],
                                      sem).wait()
            return

        @pl.when(w < _NW - 1)
        def _():
            pltpu.make_async_copy(src, out_hbm.at[pl.ds(0, _SPAN), :],
                                  sem).wait()

        @pl.when(w == _NW - 1)
        def _():
            pltpu.make_async_copy(src.at[pl.ds(0, _TAIL)],
                                  out_hbm.at[pl.ds(0, _TAIL), :],
                                  sem).wait()

    def make_issue(d, ib):
        grow = _GB * d
        srow = _B * d

        def issue(g, p):
            off = g * grow
            for t in range(_SG):
                pltpu.async_copy(
                    atoms_hbm.at[ib.at[pl.ds(off + t * srow, srow)]],
                    gbufs[p].at[pl.ds(t * srow, srow)], sgs[p])
        return issue

    def make_drain(d):
        grow = _GB * d

        def drain(p):
            pltpu.make_async_copy(atoms_hbm.at[pl.ds(0, grow), :],
                                  gbufs[p].at[pl.ds(0, grow)],
                                  sgs[p]).wait()
        return drain

    def make_compute(d, os_cur):
        def compute(g, p):
            def row_body(rr, carry):
                base = rr * d
                orow = g * _GB + rr
                for f in range(_N_FEAT // _LANES):
                    fs = pl.ds(f * _LANES, _LANES)
                    acc = os_cur[orow, fs]
                    for j in range(d):
                        acc = jnp.maximum(acc, gbufs[p][base + j, fs])
                    os_cur[orow, fs] = acc
                return carry
            lax.fori_loop(0, _GB, row_body, 0)
        return compute

    idx_copy(0, idx_bufs[0], sidx)
    self_copy(_SC_BUCKETS[0], os_bufs[0], sself)
    idx_wait(0, idx_bufs[0], sidx)
    make_issue(_SC_BUCKETS[0], idx_bufs[0])(jnp.int32(0), 0)
    self_wait(_SC_BUCKETS[0], os_bufs[0], sself)

    nsc = len(_SC_BUCKETS)
    for pos in range(nsc):
        d = _SC_BUCKETS[pos]
        os_cur = os_bufs[pos % 2]
        os_next = os_bufs[(pos + 1) % 2]
        ib = idx_bufs[pos % 2]
        issue = make_issue(d, ib)
        drain = make_drain(d)
        compute = make_compute(d, os_cur)

        if pos < nsc - 1:
            idx_copy(pos + 1, idx_bufs[(pos + 1) % 2], sidx)

        def pair_body(i, carry, issue=issue, drain=drain, compute=compute):
            issue(2 * i + 1, 1)
            drain(0)
            compute(2 * i, 0)

            @pl.when(i < _GPW // 2 - 1)
            def _():
                issue(2 * i + 2, 0)

            drain(1)
            compute(2 * i + 1, 1)
            return carry

        lax.fori_loop(0, _GPW // 2, pair_body, 0)

        if pos < nsc - 1:
            nd = _SC_BUCKETS[pos + 1]
            idx_wait(pos + 1, idx_bufs[(pos + 1) % 2], sidx)
            make_issue(nd, idx_bufs[(pos + 1) % 2])(jnp.int32(0), 0)
            if pos >= 1:
                out_wait(_SC_BUCKETS[pos - 1],
                         os_next, so[(pos - 1) % 2])
            self_copy(nd, os_next, sself)
            out_write(pos, d, os_cur, so[pos % 2])
            self_wait(nd, os_next, sself)
        else:
            out_write(pos, d, os_cur, so[pos % 2])

    out_wait(_SC_BUCKETS[nsc - 2], os_bufs[(nsc - 2) % 2], so[(nsc - 2) % 2])
    out_wait(_SC_BUCKETS[nsc - 1], os_bufs[(nsc - 1) % 2], so[(nsc - 1) % 2])


_TC_ROWS = 400


def _tc_bucket_body(idx_ref, atoms_ref, out_ref, *, d, row0=0):
    i = pl.program_id(0)
    base_self = (d - 1) * _PER_DEG + row0 + i * _TC_ROWS

    def row(r, carry):
        acc = atoms_ref[pl.ds(base_self + r, 1), :]
        for j in range(d):
            acc = jnp.maximum(
                acc, atoms_ref[pl.ds(idx_ref[0, r, j], 1), :])
        out_ref[pl.ds(r, 1), :] = acc
        return carry

    lax.fori_loop(0, _TC_ROWS, row, 0)


def _tc_bucket(atoms, idx, d, row0=0):
    import functools
    nrows = idx.shape[0]
    grid = nrows // _TC_ROWS
    return pl.pallas_call(
        functools.partial(_tc_bucket_body, d=d, row0=row0),
        grid=(grid,),
        in_specs=[
            pl.BlockSpec((1, _TC_ROWS, d), lambda i: (i, 0, 0),
                         memory_space=pltpu.SMEM),
            pl.BlockSpec((_N_ATOMS, _N_FEAT), lambda i: (0, 0)),
        ],
        out_specs=pl.BlockSpec((_TC_ROWS, _N_FEAT), lambda i: (i, 0)),
        out_shape=jax.ShapeDtypeStruct((nrows, _N_FEAT), jnp.float32),
    )(idx.reshape(grid, _TC_ROWS, d), atoms)


def kernel(atoms, deg_slice, membership, deg_adj_1, deg_adj_2, deg_adj_3,
           deg_adj_4, deg_adj_5, deg_adj_6, deg_adj_7, deg_adj_8, deg_adj_9,
           deg_adj_10):
    adjs = [deg_adj_1, deg_adj_2, deg_adj_3, deg_adj_4, deg_adj_5, deg_adj_6,
            deg_adj_7, deg_adj_8, deg_adj_9, deg_adj_10]
    idx_flats = []
    for d in _SC_BUCKETS:
        flat = adjs[d - 1].astype(jnp.int32).reshape(-1)
        pad = (_PAD_ROWS - _PER_DEG) * d
        idx_flats.append(jnp.concatenate(
            [flat, jnp.zeros((pad,), jnp.int32)]))

    mesh = plsc.VectorSubcoreMesh(core_axis_name="c", subcore_axis_name="s")
    f = pl.kernel(
        _pool_body,
        out_type=jax.ShapeDtypeStruct((len(_SC_BUCKETS) * _PER_DEG,
                                       _N_FEAT), jnp.float32),
        mesh=mesh,
        scratch_types=[
            pltpu.VMEM((_SPAN * _MAX_DEG,), jnp.int32),
            pltpu.VMEM((_SPAN * _MAX_DEG,), jnp.int32),
            pltpu.VMEM((_GB * _MAX_DEG, _N_FEAT), jnp.float32),
            pltpu.VMEM((_GB * _MAX_DEG, _N_FEAT), jnp.float32),
            pltpu.VMEM((_SPAN, _N_FEAT), jnp.float32),
            pltpu.VMEM((_SPAN, _N_FEAT), jnp.float32),
            pltpu.SemaphoreType.DMA,
            pltpu.SemaphoreType.DMA,
            pltpu.SemaphoreType.DMA,
            pltpu.SemaphoreType.DMA,
            pltpu.SemaphoreType.DMA,
            pltpu.SemaphoreType.DMA,
        ],
    )
    sc_out = f(atoms, *idx_flats)
    tc_outs = {d: _tc_bucket(atoms, adjs[d - 1].astype(jnp.int32), d)
               for d in _TC_BUCKETS}
    tc8 = _tc_bucket(atoms, adjs[7][_SC_R8:].astype(jnp.int32), 8,
                     row0=_SC_R8)
    pieces = []
    for d in range(1, _MAX_DEG + 1):
        if d in _TC_BUCKETS:
            pieces.append(tc_outs[d])
        elif d == 8:
            pos = _SC_BUCKETS.index(d)
            pieces.append(
                lax.slice_in_dim(sc_out, pos * _PER_DEG,
                                 pos * _PER_DEG + _SC_R8, axis=0))
            pieces.append(tc8)
        else:
            pos = _SC_BUCKETS.index(d)
            pieces.append(
                lax.slice_in_dim(sc_out, pos * _PER_DEG,
                                 (pos + 1) * _PER_DEG, axis=0))
    return jnp.concatenate(pieces, axis=0)


# speedup vs baseline: 1.0842x; 1.0842x over previous
import jax
import jax.numpy as jnp
from jax import lax
from jax.experimental import pallas as pl
from jax.experimental.pallas import tpu as pltpu
from jax.experimental.pallas import tpu_sc as plsc

_MAX_DEG = 10
_SC_BUCKETS = (1, 2, 3, 4, 5, 6, 7, 8)
_TC_BUCKETS = (9, 10)
_N_ATOMS = 100000
_N_FEAT = 128
_PER_DEG = 10000
_LANES = 16

_B = 8
_SG = 2
_GB = _B * _SG
_NW = 32
_GPW = 20
_SPAN = _GPW * _GB
_PAD_ROWS = _SPAN * _NW
_TAIL = _PER_DEG - (_NW - 1) * _SPAN


def _pool_body(atoms_hbm, *refs):
    nsc = len(_SC_BUCKETS)
    idx_hbms = refs[:nsc]
    out_hbm = refs[nsc]
    (idx0, idx1, gbuf0, gbuf1, osA, osB,
     sg0, sg1, sidx, sself, so0, so1) = refs[nsc + 1:]
    w = lax.axis_index("s") * 2 + lax.axis_index("c")

    idx_bufs = (idx0, idx1)
    gbufs = (gbuf0, gbuf1)
    os_bufs = (osA, osB)
    sgs = (sg0, sg1)
    so = (so0, so1)

    def idx_copy(pos, dst, sem):
        span_idx = _SPAN * _SC_BUCKETS[pos]
        pltpu.async_copy(
            idx_hbms[pos].at[pl.ds(w * span_idx, span_idx)],
            dst.at[pl.ds(0, span_idx)], sem)

    def idx_wait(pos, dst, sem):
        span_idx = _SPAN * _SC_BUCKETS[pos]
        pltpu.make_async_copy(
            idx_hbms[pos].at[pl.ds(0, span_idx)],
            dst.at[pl.ds(0, span_idx)], sem).wait()

    def self_copy(d, dst, sem):
        base_out = (d - 1) * _PER_DEG

        @pl.when(w < _NW - 1)
        def _():
            o0 = pl.multiple_of(base_out + w * _SPAN, _B)
            pltpu.async_copy(atoms_hbm.at[pl.ds(o0, _SPAN), :], dst, sem)

        @pl.when(w == _NW - 1)
        def _():
            pltpu.async_copy(
                atoms_hbm.at[pl.ds(base_out + (_NW - 1) * _SPAN, _TAIL), :],
                dst.at[pl.ds(0, _TAIL)], sem)

    def self_wait(dst, sem):
        @pl.when(w < _NW - 1)
        def _():
            pltpu.make_async_copy(atoms_hbm.at[pl.ds(0, _SPAN), :],
                                  dst, sem).wait()

        @pl.when(w == _NW - 1)
        def _():
            pltpu.make_async_copy(atoms_hbm.at[pl.ds(0, _TAIL), :],
                                  dst.at[pl.ds(0, _TAIL)], sem).wait()

    def out_write(pos, src, sem):
        base_out = pos * _PER_DEG

        @pl.when(w < _NW - 1)
        def _():
            o0 = pl.multiple_of(base_out + w * _SPAN, _B)
            pltpu.async_copy(src, out_hbm.at[pl.ds(o0, _SPAN), :], sem)

        @pl.when(w == _NW - 1)
        def _():
            pltpu.async_copy(
                src.at[pl.ds(0, _TAIL)],
                out_hbm.at[pl.ds(base_out + (_NW - 1) * _SPAN, _TAIL), :],
                sem)

    def out_wait(src, sem):
        @pl.when(w < _NW - 1)
        def _():
            pltpu.make_async_copy(src, out_hbm.at[pl.ds(0, _SPAN), :],
                                  sem).wait()

        @pl.when(w == _NW - 1)
        def _():
            pltpu.make_async_copy(src.at[pl.ds(0, _TAIL)],
                                  out_hbm.at[pl.ds(0, _TAIL), :],
                                  sem).wait()

    def make_issue(d, ib):
        grow = _GB * d
        srow = _B * d

        def issue(g, p):
            off = g * grow
            for t in range(_SG):
                pltpu.async_copy(
                    atoms_hbm.at[ib.at[pl.ds(off + t * srow, srow)]],
                    gbufs[p].at[pl.ds(t * srow, srow)], sgs[p])
        return issue

    def make_drain(d):
        grow = _GB * d

        def drain(p):
            pltpu.make_async_copy(atoms_hbm.at[pl.ds(0, grow), :],
                                  gbufs[p].at[pl.ds(0, grow)],
                                  sgs[p]).wait()
        return drain

    def make_compute(d, os_cur):
        def compute(g, p):
            def row_body(rr, carry):
                base = rr * d
                orow = g * _GB + rr
                for f in range(_N_FEAT // _LANES):
                    fs = pl.ds(f * _LANES, _LANES)
                    acc = os_cur[orow, fs]
                    for j in range(d):
                        acc = jnp.maximum(acc, gbufs[p][base + j, fs])
                    os_cur[orow, fs] = acc
                return carry
            lax.fori_loop(0, _GB, row_body, 0)
        return compute

    idx_copy(0, idx_bufs[0], sidx)
    self_copy(_SC_BUCKETS[0], os_bufs[0], sself)
    idx_wait(0, idx_bufs[0], sidx)
    make_issue(_SC_BUCKETS[0], idx_bufs[0])(jnp.int32(0), 0)
    self_wait(os_bufs[0], sself)

    nsc = len(_SC_BUCKETS)
    for pos in range(nsc):
        d = _SC_BUCKETS[pos]
        os_cur = os_bufs[pos % 2]
        os_next = os_bufs[(pos + 1) % 2]
        ib = idx_bufs[pos % 2]
        issue = make_issue(d, ib)
        drain = make_drain(d)
        compute = make_compute(d, os_cur)

        if pos < nsc - 1:
            idx_copy(pos + 1, idx_bufs[(pos + 1) % 2], sidx)

        def pair_body(i, carry, issue=issue, drain=drain, compute=compute):
            issue(2 * i + 1, 1)
            drain(0)
            compute(2 * i, 0)

            @pl.when(i < _GPW // 2 - 1)
            def _():
                issue(2 * i + 2, 0)

            drain(1)
            compute(2 * i + 1, 1)
            return carry

        lax.fori_loop(0, _GPW // 2, pair_body, 0)

        if pos < nsc - 1:
            nd = _SC_BUCKETS[pos + 1]
            idx_wait(pos + 1, idx_bufs[(pos + 1) % 2], sidx)
            make_issue(nd, idx_bufs[(pos + 1) % 2])(jnp.int32(0), 0)
            if pos >= 1:
                out_wait(os_next, so[(pos - 1) % 2])
            self_copy(nd, os_next, sself)
            out_write(pos, os_cur, so[pos % 2])
            self_wait(os_next, sself)
        else:
            out_write(pos, os_cur, so[pos % 2])

    out_wait(os_bufs[(nsc - 2) % 2], so[(nsc - 2) % 2])
    out_wait(os_bufs[(nsc - 1) % 2], so[(nsc - 1) % 2])


_TC_ROWS = 400


def _tc_bucket_body(idx_ref, atoms_ref, out_ref, *, d):
    i = pl.program_id(0)
    base_self = (d - 1) * _PER_DEG + i * _TC_ROWS

    def row(r, carry):
        acc = atoms_ref[pl.ds(base_self + r, 1), :]
        for j in range(d):
            acc = jnp.maximum(
                acc, atoms_ref[pl.ds(idx_ref[0, r, j], 1), :])
        out_ref[pl.ds(r, 1), :] = acc
        return carry

    lax.fori_loop(0, _TC_ROWS, row, 0)


def _tc_bucket(atoms, idx, d):
    import functools
    grid = _PER_DEG // _TC_ROWS
    return pl.pallas_call(
        functools.partial(_tc_bucket_body, d=d),
        grid=(grid,),
        in_specs=[
            pl.BlockSpec((1, _TC_ROWS, d), lambda i: (i, 0, 0),
                         memory_space=pltpu.SMEM),
            pl.BlockSpec((_N_ATOMS, _N_FEAT), lambda i: (0, 0)),
        ],
        out_specs=pl.BlockSpec((_TC_ROWS, _N_FEAT), lambda i: (i, 0)),
        out_shape=jax.ShapeDtypeStruct((_PER_DEG, _N_FEAT), jnp.float32),
    )(idx.reshape(_PER_DEG // _TC_ROWS, _TC_ROWS, d), atoms)


def kernel(atoms, deg_slice, membership, deg_adj_1, deg_adj_2, deg_adj_3,
           deg_adj_4, deg_adj_5, deg_adj_6, deg_adj_7, deg_adj_8, deg_adj_9,
           deg_adj_10):
    adjs = [deg_adj_1, deg_adj_2, deg_adj_3, deg_adj_4, deg_adj_5, deg_adj_6,
            deg_adj_7, deg_adj_8, deg_adj_9, deg_adj_10]
    idx_flats = []
    for d in _SC_BUCKETS:
        flat = adjs[d - 1].astype(jnp.int32).reshape(-1)
        pad = (_PAD_ROWS - _PER_DEG) * d
        idx_flats.append(jnp.concatenate(
            [flat, jnp.zeros((pad,), jnp.int32)]))

    mesh = plsc.VectorSubcoreMesh(core_axis_name="c", subcore_axis_name="s")
    f = pl.kernel(
        _pool_body,
        out_type=jax.ShapeDtypeStruct((len(_SC_BUCKETS) * _PER_DEG,
                                       _N_FEAT), jnp.float32),
        mesh=mesh,
        scratch_types=[
            pltpu.VMEM((_SPAN * _MAX_DEG,), jnp.int32),
            pltpu.VMEM((_SPAN * _MAX_DEG,), jnp.int32),
            pltpu.VMEM((_GB * _MAX_DEG, _N_FEAT), jnp.float32),
            pltpu.VMEM((_GB * _MAX_DEG, _N_FEAT), jnp.float32),
            pltpu.VMEM((_SPAN, _N_FEAT), jnp.float32),
            pltpu.VMEM((_SPAN, _N_FEAT), jnp.float32),
            pltpu.SemaphoreType.DMA,
            pltpu.SemaphoreType.DMA,
            pltpu.SemaphoreType.DMA,
            pltpu.SemaphoreType.DMA,
            pltpu.SemaphoreType.DMA,
            pltpu.SemaphoreType.DMA,
        ],
    )
    sc_out = f(atoms, *idx_flats)
    tc_outs = {d: _tc_bucket(atoms, adjs[d - 1].astype(jnp.int32), d)
               for d in _TC_BUCKETS}
    pieces = []
    for d in range(1, _MAX_DEG + 1):
        if d in _TC_BUCKETS:
            pieces.append(tc_outs[d])
        else:
            pos = _SC_BUCKETS.index(d)
            pieces.append(
                lax.slice_in_dim(sc_out, pos * _PER_DEG,
                                 (pos + 1) * _PER_DEG, axis=0))
    return jnp.concatenate(pieces, axis=0)
